# baseline (speedup 1.0000x reference)
import functools

import jax
import jax.numpy as jnp
from jax import lax
from jax.experimental import pallas as pl
from jax.experimental.pallas import tpu as pltpu
from jax.experimental.pallas import tpu_sc as plsc

VOCAB = 1000000
EMBED_DIM = 32
BATCH = 16384
HIST = 50

NC = 2
NS = 16
NW = NC * NS
NTILE = HIST * (BATCH // 128)
TILE_PER_W = NTILE // NW
NBLK_A = (VOCAB + 511) // 512
S_ROWS = NBLK_A * 128


def _table_tr_body(x_ref, o_ref):
    x3 = x_ref[...].reshape(32, 4, 128)
    for j in range(4):
        o_ref[:, j * 32:(j + 1) * 32] = x3[:, j, :].T


_table_tr = pl.pallas_call(
    _table_tr_body,
    grid=(NBLK_A,),
    in_specs=[pl.BlockSpec((32, 512), lambda c: (0, c))],
    out_specs=pl.BlockSpec((128, 128), lambda c: (c, 0)),
    out_shape=jax.ShapeDtypeStruct((S_ROWS, 128), jnp.float32),
)


def _out_tr_body(x_ref, o_ref):
    t = x_ref[0, 0].T
    t4 = t.reshape(4, 4, 8, 128)
    o_ref[0] = t4.transpose(1, 0, 2, 3)


_out_tr = pl.pallas_call(
    _out_tr_body,
    grid=(HIST, 32),
    in_specs=[pl.BlockSpec((1, 1, 128, 128), lambda h, q: (h, q, 0, 0))],
    out_specs=pl.BlockSpec((1, 4, 4, 8, 128), lambda h, q: (h, 0, q, 0, 0)),
    out_shape=jax.ShapeDtypeStruct((HIST, 4, 128, 8, 128), jnp.float32),
)


def _gather_body(tab, idx_t, out4,
                 iv0, iv1, iv2, iv3, fv0, fv1, fv2, fv3,
                 rv0, rv1, rv2, rv3,
                 sg0, sg1, sg2, sg3, so0, so1, so2, so3):
    wid = lax.axis_index("s") * NC + lax.axis_index("c")
    ivs = (iv0, iv1, iv2, iv3)
    fvs = (fv0, fv1, fv2, fv3)
    rvs = (rv0, rv1, rv2, rv3)
    sgs = (sg0, sg1, sg2, sg3)
    sos = (so0, so1, so2, so3)

    def stage(g, q):
        hb = wid * TILE_PER_W + g
        h = hb // 128
        bt = hb % 128
        pltpu.sync_copy(
            idx_t.at[h, pl.ds(pl.multiple_of(bt * 128, 128), 128)], ivs[q])
        for k in range(8):
            u = ivs[q][pl.ds(k * 16, 16)]
            f = ((u & ~511) + ((u & 127) << 2) + ((u >> 7) & 3))
            fvs[q][pl.ds(k * 16, 16)] = f

    def out_slice(g):
        hb = wid * TILE_PER_W + g
        h = hb // 128
        bt = hb % 128
        return out4.at[h, bt // 4, :, pl.ds((bt % 4) * 32, 32)]

    for q in range(4):
        stage(q, q)

    def body(p, carry):
        for q in range(4):
            g = 4 * p + q

            @pl.when(p > 0)
            def _():
                pltpu.make_async_copy(rvs[q], out_slice(g - 4), sos[q]).wait()
            pltpu.async_copy(tab.at[fvs[q]], rvs[q], sgs[q])
        for q in range(4):
            g = 4 * p + q
            pltpu.make_async_copy(tab.at[fvs[q]], rvs[q], sgs[q]).wait()
            pltpu.async_copy(rvs[q], out_slice(g), sos[q])

            @pl.when(g + 4 < NTILE // NW)
            def _():
                stage(g + 4, q)
        return carry

    lax.fori_loop(0, TILE_PER_W // 4, body, 0)
    for q in range(4):
        g = TILE_PER_W - 4 + q
        pltpu.make_async_copy(rvs[q], out_slice(g), sos[q]).wait()


_gather_call = functools.partial(
    pl.kernel,
    mesh=plsc.VectorSubcoreMesh(core_axis_name="c", subcore_axis_name="s"),
    out_type=jax.ShapeDtypeStruct((HIST, 32, 128, 128), jnp.float32),
    compiler_params=pltpu.CompilerParams(use_tc_tiling_on_sc=False),
    scratch_types=[
        pltpu.VMEM((128,), jnp.int32), pltpu.VMEM((128,), jnp.int32),
        pltpu.VMEM((128,), jnp.int32), pltpu.VMEM((128,), jnp.int32),
        pltpu.VMEM((128,), jnp.int32), pltpu.VMEM((128,), jnp.int32),
        pltpu.VMEM((128,), jnp.int32), pltpu.VMEM((128,), jnp.int32),
        pltpu.VMEM((128, EMBED_DIM), jnp.float32),
        pltpu.VMEM((128, EMBED_DIM), jnp.float32),
        pltpu.VMEM((128, EMBED_DIM), jnp.float32),
        pltpu.VMEM((128, EMBED_DIM), jnp.float32),
        pltpu.SemaphoreType.DMA, pltpu.SemaphoreType.DMA,
        pltpu.SemaphoreType.DMA, pltpu.SemaphoreType.DMA,
        pltpu.SemaphoreType.DMA, pltpu.SemaphoreType.DMA,
        pltpu.SemaphoreType.DMA, pltpu.SemaphoreType.DMA,
    ],
)(_gather_body)


@jax.jit
def kernel(inputs, table):
    s = _table_tr(table.T)
    tab_rm = s.reshape(S_ROWS * 4, EMBED_DIM)
    idx_t = inputs.T.astype(jnp.int32)
    out4 = _gather_call(tab_rm, idx_t)
    out5 = _out_tr(out4)
    return out5.transpose((2, 4, 0, 1, 3)).reshape(BATCH, HIST, EMBED_DIM)


# speedup vs baseline: 3.4431x; 3.4431x over previous
import functools

import jax
import jax.numpy as jnp
from jax import lax
from jax.experimental import pallas as pl
from jax.experimental.pallas import tpu as pltpu
from jax.experimental.pallas import tpu_sc as plsc

VOCAB = 1000000
EMBED_DIM = 32
BATCH = 16384
HIST = 50

NC = 2
NS = 16
NW = NC * NS
NQUAD = HIST * (BATCH // 512)
QUAD_PER_W = NQUAD // NW
ACOL = 4096
NBLK_A = (VOCAB + ACOL - 1) // ACOL
S_ROWS = NBLK_A * (ACOL // 4)


def _table_tr_body(x_ref, o_ref):
    x = x_ref[...]
    for sb in range(ACOL // 512):
        for j in range(4):
            o_ref[sb * 128:(sb + 1) * 128, j * 32:(j + 1) * 32] = (
                x[:, sb * 512 + j * 128: sb * 512 + (j + 1) * 128].T)


_table_tr = pl.pallas_call(
    _table_tr_body,
    grid=(NBLK_A,),
    in_specs=[pl.BlockSpec((32, ACOL), lambda c: (0, c))],
    out_specs=pl.BlockSpec((ACOL // 4, 128), lambda c: (c, 0)),
    out_shape=jax.ShapeDtypeStruct((S_ROWS, 128), jnp.float32),
)


def _out_tr_body(x_ref, o_ref):
    for k in range(4):
        t = x_ref[0, k].T
        t4 = t.reshape(4, 4, 8, 128)
        o_ref[0, :, k * 4:(k + 1) * 4] = t4.transpose(1, 0, 2, 3)


_out_tr = pl.pallas_call(
    _out_tr_body,
    grid=(HIST, 8),
    in_specs=[pl.BlockSpec((1, 4, 128, 128), lambda h, q: (h, q, 0, 0))],
    out_specs=pl.BlockSpec((1, 4, 16, 8, 128), lambda h, q: (h, 0, q, 0, 0)),
    out_shape=jax.ShapeDtypeStruct((HIST, 4, 128, 8, 128), jnp.float32),
)


def _gather_body(tab, idx_t, out4, iv0, iv1, fv0, fv1,
                 r00, r01, r02, r03, r10, r11, r12, r13,
                 sg0, sg1, so0, so1):
    wid = lax.axis_index("s") * NC + lax.axis_index("c")
    ivs = (iv0, iv1)
    fvs = (fv0, fv1)
    rvs = ((r00, r01, r02, r03), (r10, r11, r12, r13))
    sgs = (sg0, sg1)
    sos = (so0, so1)

    def stage(g, b):
        qg = wid * QUAD_PER_W + g
        h = qg // 32
        btq = qg % 32
        pltpu.sync_copy(
            idx_t.at[h, pl.ds(pl.multiple_of(btq * 512, 512), 512)], ivs[b])
        for k in range(32):
            u = ivs[b][pl.ds(k * 16, 16)]
            fvs[b][pl.ds(k * 16, 16)] = (
                (u & ~511) + ((u & 127) << 2) + ((u >> 7) & 3))

    def fire_gathers(b):
        for j in range(4):
            pltpu.async_copy(tab.at[fvs[b].at[pl.ds(j * 128, 128)]],
                             rvs[b][j], sgs[b])

    def drain_gathers(b):
        for j in range(4):
            pltpu.make_async_copy(tab.at[fvs[b].at[pl.ds(j * 128, 128)]],
                                  rvs[b][j], sgs[b]).wait()

    def out_slice(g, j):
        qg = wid * QUAD_PER_W + g
        return out4.at[qg // 32, qg % 32, :, pl.ds(j * 32, 32)]

    def fire_scatter(g, b):
        for j in range(4):
            pltpu.async_copy(rvs[b][j], out_slice(g, j), sos[b])

    def wait_scatter(g, b):
        for j in range(4):
            pltpu.make_async_copy(rvs[b][j], out_slice(g, j), sos[b]).wait()

    stage(0, 0)
    fire_gathers(0)

    def body(p, carry):
        ga = 2 * p
        gb = 2 * p + 1

        stage(gb, 1)

        @pl.when(p > 0)
        def _():
            wait_scatter(ga - 1, 1)
        fire_gathers(1)

        drain_gathers(0)
        fire_scatter(ga, 0)

        @pl.when(p + 1 < QUAD_PER_W // 2)
        def _():
            stage(ga + 2, 0)
            wait_scatter(ga, 0)
            fire_gathers(0)

        drain_gathers(1)
        fire_scatter(gb, 1)
        return carry

    lax.fori_loop(0, QUAD_PER_W // 2, body, 0)
    wait_scatter(QUAD_PER_W - 2, 0)
    wait_scatter(QUAD_PER_W - 1, 1)


_gather_call = functools.partial(
    pl.kernel,
    mesh=plsc.VectorSubcoreMesh(core_axis_name="c", subcore_axis_name="s"),
    out_type=jax.ShapeDtypeStruct((HIST, 32, 128, 128), jnp.float32),
    compiler_params=pltpu.CompilerParams(use_tc_tiling_on_sc=False),
    scratch_types=[
        pltpu.VMEM((512,), jnp.int32), pltpu.VMEM((512,), jnp.int32),
        pltpu.VMEM((512,), jnp.int32), pltpu.VMEM((512,), jnp.int32),
        pltpu.VMEM((128, EMBED_DIM), jnp.float32),
        pltpu.VMEM((128, EMBED_DIM), jnp.float32),
        pltpu.VMEM((128, EMBED_DIM), jnp.float32),
        pltpu.VMEM((128, EMBED_DIM), jnp.float32),
        pltpu.VMEM((128, EMBED_DIM), jnp.float32),
        pltpu.VMEM((128, EMBED_DIM), jnp.float32),
        pltpu.VMEM((128, EMBED_DIM), jnp.float32),
        pltpu.VMEM((128, EMBED_DIM), jnp.float32),
        pltpu.SemaphoreType.DMA, pltpu.SemaphoreType.DMA,
        pltpu.SemaphoreType.DMA, pltpu.SemaphoreType.DMA,
    ],
)(_gather_body)


@jax.jit
def kernel(inputs, table):
    s = _table_tr(table.T)
    tab_rm = s.reshape(S_ROWS * 4, EMBED_DIM)
    idx_t = inputs.T.astype(jnp.int32)
    out4 = _gather_call(tab_rm, idx_t)
    out5 = _out_tr(out4)
    return out5.transpose((2, 4, 0, 1, 3)).reshape(BATCH, HIST, EMBED_DIM)


# speedup vs baseline: 4.4453x; 1.2911x over previous
import functools

import jax
import jax.numpy as jnp
from jax import lax
from jax.experimental import pallas as pl
from jax.experimental.pallas import tpu as pltpu
from jax.experimental.pallas import tpu_sc as plsc

VOCAB = 1000000
EMBED_DIM = 32
BATCH = 16384
HIST = 50

NC = 2
NS = 16
NW = NC * NS
NQUAD = HIST * (BATCH // 512)
QUAD_PER_W = NQUAD // NW
ACOL = 8192
NBLK_A = (VOCAB + ACOL - 1) // ACOL
S_ROWS = NBLK_A * (ACOL // 4)


def _table_tr_body(x_ref, o_ref):
    x = x_ref[...]
    for sb in range(ACOL // 512):
        for j in range(4):
            o_ref[sb * 128:(sb + 1) * 128, j * 32:(j + 1) * 32] = (
                x[:, sb * 512 + j * 128: sb * 512 + (j + 1) * 128].T)


_table_tr = pl.pallas_call(
    _table_tr_body,
    grid=(NBLK_A,),
    in_specs=[pl.BlockSpec((32, ACOL), lambda c: (0, c))],
    out_specs=pl.BlockSpec((ACOL // 4, 128), lambda c: (c, 0)),
    out_shape=jax.ShapeDtypeStruct((S_ROWS, 128), jnp.float32),
)


def _out_tr_body(x_ref, o_ref):
    for k in range(8):
        t = x_ref[0, k].T
        t4 = t.reshape(4, 4, 8, 128)
        o_ref[0, :, k * 4:(k + 1) * 4] = t4.transpose(1, 0, 2, 3)


_out_tr = pl.pallas_call(
    _out_tr_body,
    grid=(HIST, 4),
    in_specs=[pl.BlockSpec((1, 8, 128, 128), lambda h, q: (h, q, 0, 0))],
    out_specs=pl.BlockSpec((1, 4, 32, 8, 128), lambda h, q: (h, 0, q, 0, 0)),
    out_shape=jax.ShapeDtypeStruct((HIST, 4, 128, 8, 128), jnp.float32),
)


def _gather_body(tab, idx_t, out4, iv0, iv1, fv0, fv1,
                 r00, r01, r02, r03, r10, r11, r12, r13,
                 sg0, sg1, so0, so1):
    wid = lax.axis_index("s") * NC + lax.axis_index("c")
    ivs = (iv0, iv1)
    fvs = (fv0, fv1)
    rvs = ((r00, r01, r02, r03), (r10, r11, r12, r13))
    sgs = (sg0, sg1)
    sos = (so0, so1)

    def stage(g, b):
        qg = wid * QUAD_PER_W + g
        h = qg // 32
        btq = qg % 32
        pltpu.sync_copy(
            idx_t.at[h, pl.ds(pl.multiple_of(btq * 512, 512), 512)], ivs[b])
        for k in range(32):
            u = ivs[b][pl.ds(k * 16, 16)]
            fvs[b][pl.ds(k * 16, 16)] = (
                (u & ~511) + ((u & 127) << 2) + ((u >> 7) & 3))

    def fire_gathers(b):
        for j in range(4):
            pltpu.async_copy(tab.at[fvs[b].at[pl.ds(j * 128, 128)]],
                             rvs[b][j], sgs[b])

    def drain_gathers(b):
        for j in range(4):
            pltpu.make_async_copy(tab.at[fvs[b].at[pl.ds(j * 128, 128)]],
                                  rvs[b][j], sgs[b]).wait()

    def out_slice(g, j):
        qg = wid * QUAD_PER_W + g
        return out4.at[qg // 32, qg % 32, :, pl.ds(j * 32, 32)]

    def fire_scatter(g, b):
        for j in range(4):
            pltpu.async_copy(rvs[b][j], out_slice(g, j), sos[b])

    def wait_scatter(g, b):
        for j in range(4):
            pltpu.make_async_copy(rvs[b][j], out_slice(g, j), sos[b]).wait()

    stage(0, 0)
    fire_gathers(0)

    def body(p, carry):
        ga = 2 * p
        gb = 2 * p + 1

        stage(gb, 1)

        @pl.when(p > 0)
        def _():
            wait_scatter(ga - 1, 1)
        fire_gathers(1)

        drain_gathers(0)
        fire_scatter(ga, 0)

        @pl.when(p + 1 < QUAD_PER_W // 2)
        def _():
            stage(ga + 2, 0)
            wait_scatter(ga, 0)
            fire_gathers(0)

        drain_gathers(1)
        fire_scatter(gb, 1)
        return carry

    lax.fori_loop(0, QUAD_PER_W // 2, body, 0)
    wait_scatter(QUAD_PER_W - 2, 0)
    wait_scatter(QUAD_PER_W - 1, 1)


_gather_call = functools.partial(
    pl.kernel,
    mesh=plsc.VectorSubcoreMesh(core_axis_name="c", subcore_axis_name="s"),
    out_type=jax.ShapeDtypeStruct((HIST, 32, 128, 128), jnp.float32),
    compiler_params=pltpu.CompilerParams(use_tc_tiling_on_sc=False),
    scratch_types=[
        pltpu.VMEM((512,), jnp.int32), pltpu.VMEM((512,), jnp.int32),
        pltpu.VMEM((512,), jnp.int32), pltpu.VMEM((512,), jnp.int32),
        pltpu.VMEM((128, EMBED_DIM), jnp.float32),
        pltpu.VMEM((128, EMBED_DIM), jnp.float32),
        pltpu.VMEM((128, EMBED_DIM), jnp.float32),
        pltpu.VMEM((128, EMBED_DIM), jnp.float32),
        pltpu.VMEM((128, EMBED_DIM), jnp.float32),
        pltpu.VMEM((128, EMBED_DIM), jnp.float32),
        pltpu.VMEM((128, EMBED_DIM), jnp.float32),
        pltpu.VMEM((128, EMBED_DIM), jnp.float32),
        pltpu.SemaphoreType.DMA, pltpu.SemaphoreType.DMA,
        pltpu.SemaphoreType.DMA, pltpu.SemaphoreType.DMA,
    ],
)(_gather_body)


@jax.jit
def kernel(inputs, table):
    s = _table_tr(table.T)
    tab_rm = s.reshape(S_ROWS * 4, EMBED_DIM)
    idx_t = inputs.T.astype(jnp.int32)
    out4 = _gather_call(tab_rm, idx_t)
    out5 = _out_tr(out4)
    return out5.transpose((2, 4, 0, 1, 3)).reshape(BATCH, HIST, EMBED_DIM)


# speedup vs baseline: 4.9243x; 1.1078x over previous
import functools

import jax
import jax.numpy as jnp
from jax import lax
from jax.experimental import pallas as pl
from jax.experimental.pallas import tpu as pltpu
from jax.experimental.pallas import tpu_sc as plsc

VOCAB = 1000000
EMBED_DIM = 32
BATCH = 16384
HIST = 50

NC = 2
NS = 16
NW = NC * NS
NQUAD = HIST * (BATCH // 512)
QUAD_PER_W = NQUAD // NW
ACOL = 16384
NBLK_A = (VOCAB + ACOL - 1) // ACOL
S_ROWS = NBLK_A * (ACOL // 4)


def _table_tr_body(x_ref, o_ref):
    x = x_ref[...]
    for sb in range(ACOL // 512):
        for j in range(4):
            o_ref[sb * 128:(sb + 1) * 128, j * 32:(j + 1) * 32] = (
                x[:, sb * 512 + j * 128: sb * 512 + (j + 1) * 128].T)


_table_tr = pl.pallas_call(
    _table_tr_body,
    grid=(NBLK_A,),
    in_specs=[pl.BlockSpec((32, ACOL), lambda c: (0, c))],
    out_specs=pl.BlockSpec((ACOL // 4, 128), lambda c: (c, 0)),
    out_shape=jax.ShapeDtypeStruct((S_ROWS, 128), jnp.float32),
)


def _out_tr_body(x_ref, o_ref):
    for k in range(16):
        t = x_ref[0, k].T
        t4 = t.reshape(4, 4, 8, 128)
        o_ref[0, :, k * 4:(k + 1) * 4] = t4.transpose(1, 0, 2, 3)


_out_tr = pl.pallas_call(
    _out_tr_body,
    grid=(HIST, 2),
    in_specs=[pl.BlockSpec((1, 16, 128, 128), lambda h, q: (h, q, 0, 0))],
    out_specs=pl.BlockSpec((1, 4, 64, 8, 128), lambda h, q: (h, 0, q, 0, 0)),
    out_shape=jax.ShapeDtypeStruct((HIST, 4, 128, 8, 128), jnp.float32),
)


def _gather_body(tab, idx_t, out4, iv0, iv1, fv0, fv1,
                 r00, r01, r02, r03, r10, r11, r12, r13,
                 sg0, sg1, so0, so1):
    wid = lax.axis_index("s") * NC + lax.axis_index("c")
    ivs = (iv0, iv1)
    fvs = (fv0, fv1)
    rvs = ((r00, r01, r02, r03), (r10, r11, r12, r13))
    sgs = (sg0, sg1)
    sos = (so0, so1)

    def stage(g, b):
        qg = wid * QUAD_PER_W + g
        h = qg // 32
        btq = qg % 32
        pltpu.sync_copy(
            idx_t.at[h, pl.ds(pl.multiple_of(btq * 512, 512), 512)], ivs[b])
        for k in range(32):
            u = ivs[b][pl.ds(k * 16, 16)]
            fvs[b][pl.ds(k * 16, 16)] = (
                (u & ~511) + ((u & 127) << 2) + ((u >> 7) & 3))

    def fire_gathers(b):
        for j in range(4):
            pltpu.async_copy(tab.at[fvs[b].at[pl.ds(j * 128, 128)]],
                             rvs[b][j], sgs[b])

    def drain_gathers(b):
        for j in range(4):
            pltpu.make_async_copy(tab.at[fvs[b].at[pl.ds(j * 128, 128)]],
                                  rvs[b][j], sgs[b]).wait()

    def out_slice(g, j):
        qg = wid * QUAD_PER_W + g
        return out4.at[qg // 32, qg % 32, :, pl.ds(j * 32, 32)]

    def fire_scatter(g, b):
        for j in range(4):
            pltpu.async_copy(rvs[b][j], out_slice(g, j), sos[b])

    def wait_scatter(g, b):
        for j in range(4):
            pltpu.make_async_copy(rvs[b][j], out_slice(g, j), sos[b]).wait()

    stage(0, 0)
    fire_gathers(0)

    def body(p, carry):
        ga = 2 * p
        gb = 2 * p + 1

        stage(gb, 1)

        @pl.when(p > 0)
        def _():
            wait_scatter(ga - 1, 1)
        fire_gathers(1)

        drain_gathers(0)
        fire_scatter(ga, 0)

        @pl.when(p + 1 < QUAD_PER_W // 2)
        def _():
            stage(ga + 2, 0)
            wait_scatter(ga, 0)
            fire_gathers(0)

        drain_gathers(1)
        fire_scatter(gb, 1)
        return carry

    lax.fori_loop(0, QUAD_PER_W // 2, body, 0)
    wait_scatter(QUAD_PER_W - 2, 0)
    wait_scatter(QUAD_PER_W - 1, 1)


_gather_call = functools.partial(
    pl.kernel,
    mesh=plsc.VectorSubcoreMesh(core_axis_name="c", subcore_axis_name="s"),
    out_type=jax.ShapeDtypeStruct((HIST, 32, 128, 128), jnp.float32),
    compiler_params=pltpu.CompilerParams(use_tc_tiling_on_sc=False),
    scratch_types=[
        pltpu.VMEM((512,), jnp.int32), pltpu.VMEM((512,), jnp.int32),
        pltpu.VMEM((512,), jnp.int32), pltpu.VMEM((512,), jnp.int32),
        pltpu.VMEM((128, EMBED_DIM), jnp.float32),
        pltpu.VMEM((128, EMBED_DIM), jnp.float32),
        pltpu.VMEM((128, EMBED_DIM), jnp.float32),
        pltpu.VMEM((128, EMBED_DIM), jnp.float32),
        pltpu.VMEM((128, EMBED_DIM), jnp.float32),
        pltpu.VMEM((128, EMBED_DIM), jnp.float32),
        pltpu.VMEM((128, EMBED_DIM), jnp.float32),
        pltpu.VMEM((128, EMBED_DIM), jnp.float32),
        pltpu.SemaphoreType.DMA, pltpu.SemaphoreType.DMA,
        pltpu.SemaphoreType.DMA, pltpu.SemaphoreType.DMA,
    ],
)(_gather_body)


@jax.jit
def kernel(inputs, table):
    s = _table_tr(table.T)
    tab_rm = s.reshape(S_ROWS * 4, EMBED_DIM)
    idx_t = inputs.T.astype(jnp.int32)
    out4 = _gather_call(tab_rm, idx_t)
    out5 = _out_tr(out4)
    return out5.transpose((2, 4, 0, 1, 3)).reshape(BATCH, HIST, EMBED_DIM)


# speedup vs baseline: 5.2922x; 1.0747x over previous
import functools

import jax
import jax.numpy as jnp
from jax import lax
from jax.experimental import pallas as pl
from jax.experimental.pallas import tpu as pltpu
from jax.experimental.pallas import tpu_sc as plsc

VOCAB = 1000000
EMBED_DIM = 32
BATCH = 16384
HIST = 50

NC = 2
NS = 16
NW = NC * NS
NQUAD = HIST * (BATCH // 512)
QUAD_PER_W = NQUAD // NW
ACOL = 32768
NBLK_A = (VOCAB + ACOL - 1) // ACOL
S_ROWS = NBLK_A * (ACOL // 4)


def _table_tr_body(x_ref, o_ref):
    x = x_ref[...]
    for sb in range(ACOL // 512):
        for j in range(4):
            o_ref[sb * 128:(sb + 1) * 128, j * 32:(j + 1) * 32] = (
                x[:, sb * 512 + j * 128: sb * 512 + (j + 1) * 128].T)


_table_tr = pl.pallas_call(
    _table_tr_body,
    grid=(NBLK_A,),
    in_specs=[pl.BlockSpec((32, ACOL), lambda c: (0, c))],
    out_specs=pl.BlockSpec((ACOL // 4, 128), lambda c: (c, 0)),
    out_shape=jax.ShapeDtypeStruct((S_ROWS, 128), jnp.float32),
)


def _out_tr_body(x_ref, o_ref):
    for k in range(32):
        t = x_ref[0, k].T
        t4 = t.reshape(4, 4, 8, 128)
        o_ref[0, :, k * 4:(k + 1) * 4] = t4.transpose(1, 0, 2, 3)


_out_tr = pl.pallas_call(
    _out_tr_body,
    grid=(HIST,),
    in_specs=[pl.BlockSpec((1, 32, 128, 128), lambda h: (h, 0, 0, 0))],
    out_specs=pl.BlockSpec((1, 4, 128, 8, 128), lambda h: (h, 0, 0, 0, 0)),
    out_shape=jax.ShapeDtypeStruct((HIST, 4, 128, 8, 128), jnp.float32),
)


def _gather_body(tab, idx_t, out4, iv0, iv1, fv0, fv1,
                 r00, r01, r02, r03, r10, r11, r12, r13,
                 sg0, sg1, so0, so1):
    wid = lax.axis_index("s") * NC + lax.axis_index("c")
    ivs = (iv0, iv1)
    fvs = (fv0, fv1)
    rvs = ((r00, r01, r02, r03), (r10, r11, r12, r13))
    sgs = (sg0, sg1)
    sos = (so0, so1)

    def stage(g, b):
        qg = wid * QUAD_PER_W + g
        h = qg // 32
        btq = qg % 32
        pltpu.sync_copy(
            idx_t.at[h, pl.ds(pl.multiple_of(btq * 512, 512), 512)], ivs[b])
        for k in range(32):
            u = ivs[b][pl.ds(k * 16, 16)]
            fvs[b][pl.ds(k * 16, 16)] = (
                (u & ~511) + ((u & 127) << 2) + ((u >> 7) & 3))

    def fire_gathers(b):
        for j in range(4):
            pltpu.async_copy(tab.at[fvs[b].at[pl.ds(j * 128, 128)]],
                             rvs[b][j], sgs[b])

    def drain_gathers(b):
        for j in range(4):
            pltpu.make_async_copy(tab.at[fvs[b].at[pl.ds(j * 128, 128)]],
                                  rvs[b][j], sgs[b]).wait()

    def out_slice(g, j):
        qg = wid * QUAD_PER_W + g
        return out4.at[qg // 32, qg % 32, :, pl.ds(j * 32, 32)]

    def fire_scatter(g, b):
        for j in range(4):
            pltpu.async_copy(rvs[b][j], out_slice(g, j), sos[b])

    def wait_scatter(g, b):
        for j in range(4):
            pltpu.make_async_copy(rvs[b][j], out_slice(g, j), sos[b]).wait()

    stage(0, 0)
    fire_gathers(0)

    def body(p, carry):
        ga = 2 * p
        gb = 2 * p + 1

        stage(gb, 1)

        @pl.when(p > 0)
        def _():
            wait_scatter(ga - 1, 1)
        fire_gathers(1)

        drain_gathers(0)
        fire_scatter(ga, 0)

        @pl.when(p + 1 < QUAD_PER_W // 2)
        def _():
            stage(ga + 2, 0)
            wait_scatter(ga, 0)
            fire_gathers(0)

        drain_gathers(1)
        fire_scatter(gb, 1)
        return carry

    lax.fori_loop(0, QUAD_PER_W // 2, body, 0)
    wait_scatter(QUAD_PER_W - 2, 0)
    wait_scatter(QUAD_PER_W - 1, 1)


_gather_call = functools.partial(
    pl.kernel,
    mesh=plsc.VectorSubcoreMesh(core_axis_name="c", subcore_axis_name="s"),
    out_type=jax.ShapeDtypeStruct((HIST, 32, 128, 128), jnp.float32),
    compiler_params=pltpu.CompilerParams(use_tc_tiling_on_sc=False),
    scratch_types=[
        pltpu.VMEM((512,), jnp.int32), pltpu.VMEM((512,), jnp.int32),
        pltpu.VMEM((512,), jnp.int32), pltpu.VMEM((512,), jnp.int32),
        pltpu.VMEM((128, EMBED_DIM), jnp.float32),
        pltpu.VMEM((128, EMBED_DIM), jnp.float32),
        pltpu.VMEM((128, EMBED_DIM), jnp.float32),
        pltpu.VMEM((128, EMBED_DIM), jnp.float32),
        pltpu.VMEM((128, EMBED_DIM), jnp.float32),
        pltpu.VMEM((128, EMBED_DIM), jnp.float32),
        pltpu.VMEM((128, EMBED_DIM), jnp.float32),
        pltpu.VMEM((128, EMBED_DIM), jnp.float32),
        pltpu.SemaphoreType.DMA, pltpu.SemaphoreType.DMA,
        pltpu.SemaphoreType.DMA, pltpu.SemaphoreType.DMA,
    ],
)(_gather_body)


@jax.jit
def kernel(inputs, table):
    s = _table_tr(table.T)
    tab_rm = s.reshape(S_ROWS * 4, EMBED_DIM)
    idx_t = inputs.T.astype(jnp.int32)
    out4 = _gather_call(tab_rm, idx_t)
    out5 = _out_tr(out4)
    return out5.transpose((2, 4, 0, 1, 3)).reshape(BATCH, HIST, EMBED_DIM)


# speedup vs baseline: 5.3686x; 1.0144x over previous
import functools

import jax
import jax.numpy as jnp
from jax import lax
from jax.experimental import pallas as pl
from jax.experimental.pallas import tpu as pltpu
from jax.experimental.pallas import tpu_sc as plsc

VOCAB = 1000000
EMBED_DIM = 32
BATCH = 16384
HIST = 50

NC = 2
NS = 16
NW = NC * NS
NQUAD = HIST * (BATCH // 512)
QUAD_PER_W = NQUAD // NW
ACOL = 65536
NBLK_A = (VOCAB + ACOL - 1) // ACOL
S_ROWS = NBLK_A * (ACOL // 4)


def _table_tr_body(x_ref, o_ref):
    x = x_ref[...]
    for sb in range(ACOL // 512):
        for j in range(4):
            o_ref[sb * 128:(sb + 1) * 128, j * 32:(j + 1) * 32] = (
                x[:, sb * 512 + j * 128: sb * 512 + (j + 1) * 128].T)


_table_tr = pl.pallas_call(
    _table_tr_body,
    grid=(NBLK_A,),
    in_specs=[pl.BlockSpec((32, ACOL), lambda c: (0, c))],
    out_specs=pl.BlockSpec((ACOL // 4, 128), lambda c: (c, 0)),
    out_shape=jax.ShapeDtypeStruct((S_ROWS, 128), jnp.float32),
)


def _out_tr_body(x_ref, o_ref):
    for hh in range(2):
        for k in range(32):
            t = x_ref[hh, k].T
            t4 = t.reshape(4, 4, 8, 128)
            o_ref[hh, :, k * 4:(k + 1) * 4] = t4.transpose(1, 0, 2, 3)


_out_tr = pl.pallas_call(
    _out_tr_body,
    grid=(HIST // 2,),
    in_specs=[pl.BlockSpec((2, 32, 128, 128), lambda h: (h, 0, 0, 0))],
    out_specs=pl.BlockSpec((2, 4, 128, 8, 128), lambda h: (h, 0, 0, 0, 0)),
    out_shape=jax.ShapeDtypeStruct((HIST, 4, 128, 8, 128), jnp.float32),
)


def _gather_body(tab, idx_t, out4, iv0, iv1, fv0, fv1,
                 r00, r01, r02, r03, r10, r11, r12, r13,
                 sg0, sg1, so0, so1):
    wid = lax.axis_index("s") * NC + lax.axis_index("c")
    ivs = (iv0, iv1)
    fvs = (fv0, fv1)
    rvs = ((r00, r01, r02, r03), (r10, r11, r12, r13))
    sgs = (sg0, sg1)
    sos = (so0, so1)

    def stage(g, b):
        qg = wid * QUAD_PER_W + g
        h = qg // 32
        btq = qg % 32
        pltpu.sync_copy(
            idx_t.at[h, pl.ds(pl.multiple_of(btq * 512, 512), 512)], ivs[b])
        for k in range(32):
            u = ivs[b][pl.ds(k * 16, 16)]
            fvs[b][pl.ds(k * 16, 16)] = (
                (u & ~511) + ((u & 127) << 2) + ((u >> 7) & 3))

    def fire_gathers(b):
        for j in range(4):
            pltpu.async_copy(tab.at[fvs[b].at[pl.ds(j * 128, 128)]],
                             rvs[b][j], sgs[b])

    def drain_gathers(b):
        for j in range(4):
            pltpu.make_async_copy(tab.at[fvs[b].at[pl.ds(j * 128, 128)]],
                                  rvs[b][j], sgs[b]).wait()

    def out_slice(g, j):
        qg = wid * QUAD_PER_W + g
        return out4.at[qg // 32, qg % 32, :, pl.ds(j * 32, 32)]

    def fire_scatter(g, b):
        for j in range(4):
            pltpu.async_copy(rvs[b][j], out_slice(g, j), sos[b])

    def wait_scatter(g, b):
        for j in range(4):
            pltpu.make_async_copy(rvs[b][j], out_slice(g, j), sos[b]).wait()

    stage(0, 0)
    fire_gathers(0)

    def body(p, carry):
        ga = 2 * p
        gb = 2 * p + 1

        stage(gb, 1)

        @pl.when(p > 0)
        def _():
            wait_scatter(ga - 1, 1)
        fire_gathers(1)

        drain_gathers(0)
        fire_scatter(ga, 0)

        @pl.when(p + 1 < QUAD_PER_W // 2)
        def _():
            stage(ga + 2, 0)
            wait_scatter(ga, 0)
            fire_gathers(0)

        drain_gathers(1)
        fire_scatter(gb, 1)
        return carry

    lax.fori_loop(0, QUAD_PER_W // 2, body, 0)
    wait_scatter(QUAD_PER_W - 2, 0)
    wait_scatter(QUAD_PER_W - 1, 1)


_gather_call = functools.partial(
    pl.kernel,
    mesh=plsc.VectorSubcoreMesh(core_axis_name="c", subcore_axis_name="s"),
    out_type=jax.ShapeDtypeStruct((HIST, 32, 128, 128), jnp.float32),
    compiler_params=pltpu.CompilerParams(use_tc_tiling_on_sc=False),
    scratch_types=[
        pltpu.VMEM((512,), jnp.int32), pltpu.VMEM((512,), jnp.int32),
        pltpu.VMEM((512,), jnp.int32), pltpu.VMEM((512,), jnp.int32),
        pltpu.VMEM((128, EMBED_DIM), jnp.float32),
        pltpu.VMEM((128, EMBED_DIM), jnp.float32),
        pltpu.VMEM((128, EMBED_DIM), jnp.float32),
        pltpu.VMEM((128, EMBED_DIM), jnp.float32),
        pltpu.VMEM((128, EMBED_DIM), jnp.float32),
        pltpu.VMEM((128, EMBED_DIM), jnp.float32),
        pltpu.VMEM((128, EMBED_DIM), jnp.float32),
        pltpu.VMEM((128, EMBED_DIM), jnp.float32),
        pltpu.SemaphoreType.DMA, pltpu.SemaphoreType.DMA,
        pltpu.SemaphoreType.DMA, pltpu.SemaphoreType.DMA,
    ],
)(_gather_body)


@jax.jit
def kernel(inputs, table):
    s = _table_tr(table.T)
    tab_rm = s.reshape(S_ROWS * 4, EMBED_DIM)
    idx_t = inputs.T.astype(jnp.int32)
    out4 = _gather_call(tab_rm, idx_t)
    out5 = _out_tr(out4)
    return out5.transpose((2, 4, 0, 1, 3)).reshape(BATCH, HIST, EMBED_DIM)
